# baseline (speedup 1.0000x reference)
import jax
import jax.numpy as jnp
from jax.experimental import pallas as pl
from jax.experimental.pallas import tpu as pltpu

_EPS = 1e-12
_BS = 256
_VMEM_LIMIT = 120 * 1024 * 1024


def _emb_ln_kernel(ids_ref, words_ref, pos_ref, tt_ref, gamma_ref, beta_ref,
                   out_ref):
    x = words_ref[...] + pos_ref[...][None, :, :]
    idf = ids_ref[...].astype(jnp.float32)
    tt0 = tt_ref[0][None, None, :]
    diff = (tt_ref[1] - tt_ref[0])[None, None, :]
    x = x + tt0 + idf * diff
    h = x.shape[-1]
    mu = jnp.sum(x, axis=-1, keepdims=True) * (1.0 / h)
    xc = x - mu
    var = jnp.sum(xc * xc, axis=-1, keepdims=True) * (1.0 / h)
    y = xc * jax.lax.rsqrt(var + _EPS)
    out_ref[...] = y * gamma_ref[...][None, :, :] + beta_ref[...][None, :, :]


def kernel(words_embeddings, token_type_ids, position_table,
           token_type_table, ln_gamma, ln_beta):
    b, s, h = words_embeddings.shape
    bs = min(_BS, s)
    ids3 = token_type_ids.astype(jnp.int32).reshape(b, s, 1)
    gamma2 = ln_gamma.reshape(1, h)
    beta2 = ln_beta.reshape(1, h)

    grid = (s // bs,)
    return pl.pallas_call(
        _emb_ln_kernel,
        grid=grid,
        in_specs=[
            pl.BlockSpec((b, bs, 1), lambda i: (0, i, 0)),
            pl.BlockSpec((b, bs, h), lambda i: (0, i, 0)),
            pl.BlockSpec((bs, h), lambda i: (i, 0)),
            pl.BlockSpec(token_type_table.shape, lambda i: (0, 0)),
            pl.BlockSpec((1, h), lambda i: (0, 0)),
            pl.BlockSpec((1, h), lambda i: (0, 0)),
        ],
        out_specs=pl.BlockSpec((b, bs, h), lambda i: (0, i, 0)),
        out_shape=jax.ShapeDtypeStruct((b, s, h), jnp.float32),
        compiler_params=pltpu.CompilerParams(
            dimension_semantics=("parallel",),
            vmem_limit_bytes=_VMEM_LIMIT,
        ),
    )(ids3, words_embeddings, position_table, token_type_table, gamma2,
      beta2)


# speedup vs baseline: 1.0271x; 1.0271x over previous
import jax
import jax.numpy as jnp
from jax.experimental import pallas as pl
from jax.experimental.pallas import tpu as pltpu

_EPS = 1e-12
_BS = 512
_VMEM_LIMIT = 120 * 1024 * 1024


def _emb_ln_kernel(ids_ref, words_ref, pos_ref, tt_ref, gamma_ref, beta_ref,
                   out_ref):
    x = words_ref[...] + pos_ref[...][None, :, :]
    idf = ids_ref[...].astype(jnp.float32)
    tt0 = tt_ref[0][None, None, :]
    diff = (tt_ref[1] - tt_ref[0])[None, None, :]
    x = x + tt0 + idf * diff
    h = x.shape[-1]
    mu = jnp.sum(x, axis=-1, keepdims=True) * (1.0 / h)
    xc = x - mu
    var = jnp.sum(xc * xc, axis=-1, keepdims=True) * (1.0 / h)
    y = xc * jax.lax.rsqrt(var + _EPS)
    out_ref[...] = y * gamma_ref[...][None, :, :] + beta_ref[...][None, :, :]


def kernel(words_embeddings, token_type_ids, position_table,
           token_type_table, ln_gamma, ln_beta):
    b, s, h = words_embeddings.shape
    bs = min(_BS, s)
    ids3 = token_type_ids.astype(jnp.int32).reshape(b, s, 1)
    gamma2 = ln_gamma.reshape(1, h)
    beta2 = ln_beta.reshape(1, h)

    grid = (s // bs,)
    return pl.pallas_call(
        _emb_ln_kernel,
        grid=grid,
        in_specs=[
            pl.BlockSpec((b, bs, 1), lambda i: (0, i, 0)),
            pl.BlockSpec((b, bs, h), lambda i: (0, i, 0)),
            pl.BlockSpec((bs, h), lambda i: (i, 0)),
            pl.BlockSpec(token_type_table.shape, lambda i: (0, 0)),
            pl.BlockSpec((1, h), lambda i: (0, 0)),
            pl.BlockSpec((1, h), lambda i: (0, 0)),
        ],
        out_specs=pl.BlockSpec((b, bs, h), lambda i: (0, i, 0)),
        out_shape=jax.ShapeDtypeStruct((b, s, h), jnp.float32),
        compiler_params=pltpu.CompilerParams(
            dimension_semantics=("parallel",),
            vmem_limit_bytes=_VMEM_LIMIT,
        ),
    )(ids3, words_embeddings, position_table, token_type_table, gamma2,
      beta2)


# speedup vs baseline: 1.0293x; 1.0022x over previous
import jax
import jax.numpy as jnp
from jax.experimental import pallas as pl
from jax.experimental.pallas import tpu as pltpu

_EPS = 1e-12
_BS = 512
_VMEM_LIMIT = 120 * 1024 * 1024


def _emb_ln_kernel(ids_ref, words_ref, pos_ref, tt_ref, gamma_ref, beta_ref,
                   out_ref):
    posc = pos_ref[...] + tt_ref[0][None, :]
    idf = ids_ref[...].astype(jnp.float32)
    diff = (tt_ref[1] - tt_ref[0])[None, None, :]
    x = (words_ref[...] + posc[None, :, :]) + idf * diff
    h = x.shape[-1]
    mu = jnp.sum(x, axis=-1, keepdims=True) * (1.0 / h)
    xc = x - mu
    var = jnp.sum(xc * xc, axis=-1, keepdims=True) * (1.0 / h)
    y = xc * jax.lax.rsqrt(var + _EPS)
    out_ref[...] = y * gamma_ref[...][None, :, :] + beta_ref[...][None, :, :]


def kernel(words_embeddings, token_type_ids, position_table,
           token_type_table, ln_gamma, ln_beta):
    b, s, h = words_embeddings.shape
    bs = min(_BS, s)
    ids3 = token_type_ids.astype(jnp.int32).reshape(b, s, 1)
    gamma2 = ln_gamma.reshape(1, h)
    beta2 = ln_beta.reshape(1, h)

    grid = (s // bs,)
    return pl.pallas_call(
        _emb_ln_kernel,
        grid=grid,
        in_specs=[
            pl.BlockSpec((b, bs, 1), lambda i: (0, i, 0)),
            pl.BlockSpec((b, bs, h), lambda i: (0, i, 0)),
            pl.BlockSpec((bs, h), lambda i: (i, 0)),
            pl.BlockSpec(token_type_table.shape, lambda i: (0, 0)),
            pl.BlockSpec((1, h), lambda i: (0, 0)),
            pl.BlockSpec((1, h), lambda i: (0, 0)),
        ],
        out_specs=pl.BlockSpec((b, bs, h), lambda i: (0, i, 0)),
        out_shape=jax.ShapeDtypeStruct((b, s, h), jnp.float32),
        compiler_params=pltpu.CompilerParams(
            dimension_semantics=("parallel",),
            vmem_limit_bytes=_VMEM_LIMIT,
        ),
    )(ids3, words_embeddings, position_table, token_type_table, gamma2,
      beta2)
